# baseline (speedup 1.0000x reference)
import jax
import jax.numpy as jnp
from jax.experimental import pallas as pl
from jax.experimental.pallas import tpu as pltpu

_BLK = 200
_NBUF = 4


def _gcn_kernel(seq_ref, w_ref, adj_hbm, bias_ref, alpha_ref, out_ref,
                fts_ref, bufs_ref, sems):
    nblk = adj_hbm.shape[0] // _BLK

    def _copy(i, slot):
        return pltpu.make_async_copy(
            adj_hbm.at[pl.ds(i * _BLK, _BLK), :],
            bufs_ref.at[slot],
            sems.at[slot],
        )

    for k in range(_NBUF):
        _copy(k, k).start()

    fts_ref[...] = jax.lax.dot_general(
        seq_ref[...], w_ref[...],
        dimension_numbers=(((1,), (1,)), ((), ())),
        preferred_element_type=jnp.float32,
    )

    alpha = alpha_ref[0]

    def _body(i, carry):
        slot = jax.lax.rem(i, _NBUF)
        _copy(i, slot).wait()
        acc = jax.lax.dot_general(
            bufs_ref[slot], fts_ref[...],
            dimension_numbers=(((1,), (0,)), ((), ())),
            preferred_element_type=jnp.float32,
        )
        acc = acc + bias_ref[...]
        out_ref[pl.ds(i * _BLK, _BLK), :] = jnp.where(acc > 0, acc, alpha * acc)
        nxt = i + _NBUF

        @pl.when(nxt < nblk)
        def _():
            _copy(nxt, slot).start()

        return carry

    jax.lax.fori_loop(0, nblk, _body, 0)


def kernel(seq, adj, W, bias, alpha):
    _, n, d_in = seq.shape
    d_out = W.shape[0]
    seq2 = seq.reshape(n, d_in)
    adj2 = adj.reshape(n, n)
    bias2 = bias.reshape(1, d_out)
    alpha2 = alpha.reshape(1)

    out = pl.pallas_call(
        _gcn_kernel,
        in_specs=[
            pl.BlockSpec(memory_space=pltpu.VMEM),
            pl.BlockSpec(memory_space=pltpu.VMEM),
            pl.BlockSpec(memory_space=pl.ANY),
            pl.BlockSpec(memory_space=pltpu.VMEM),
            pl.BlockSpec(memory_space=pltpu.SMEM),
        ],
        out_specs=pl.BlockSpec(memory_space=pltpu.VMEM),
        out_shape=jax.ShapeDtypeStruct((n, d_out), jnp.float32),
        scratch_shapes=[
            pltpu.VMEM((n, d_out), jnp.float32),
            pltpu.VMEM((_NBUF, _BLK, n), jnp.float32),
            pltpu.SemaphoreType.DMA((_NBUF,)),
        ],
    )(seq2, W, adj2, bias2, alpha2)
    return out.reshape(1, n, d_out)
